# Initial kernel scaffold; baseline (speedup 1.0000x reference)
import jax
import jax.numpy as jnp
from jax.experimental import pallas as pl


def kernel(pairs, label, neighbor_relations, neighbor_tails, user_mat, entity_mat, relation_mat, W, b):
    raise NotImplementedError("write your pallas kernel here")


# speedup vs baseline: 12.4721x; 12.4721x over previous
import functools

import jax
import jax.numpy as jnp
from jax import lax
from jax.experimental import pallas as pl
from jax.experimental.pallas import tpu as pltpu
from jax.experimental.pallas import tpu_sc as plsc

DIM = 32
NN = 20
B = 16384
NC = 2
NS = 16
NW = NC * NS

IPC = 128
CHUNK = 1024
JPC = CHUNK // IPC

TOT_NBR = B * NN
NBR_PER_W = TOT_NBR // NW
NBR_CHUNKS = NBR_PER_W // CHUNK
PAIR_PER_W = B // NW


def _sc_gather(tail_idx, rel_idx, item_idx, user_idx,
               entity_mat, relation_mat, user_mat):
  mesh = plsc.VectorSubcoreMesh(core_axis_name="c", subcore_axis_name="s")

  @functools.partial(
      pl.kernel, mesh=mesh,
      out_type=[
          jax.ShapeDtypeStruct((TOT_NBR, DIM), jnp.float32),
          jax.ShapeDtypeStruct((TOT_NBR, DIM), jnp.float32),
          jax.ShapeDtypeStruct((B, DIM), jnp.float32),
          jax.ShapeDtypeStruct((B, DIM), jnp.float32),
      ],
      scratch_types=[
          pltpu.VMEM((JPC, IPC), jnp.int32),
          pltpu.VMEM((CHUNK, DIM), jnp.float32),
          pltpu.SemaphoreType.DMA,
      ],
      compiler_params=pltpu.CompilerParams(use_tc_tiling_on_sc=False),
  )
  def k(tail_idx_h, rel_idx_h, item_idx_h, user_idx_h,
        entity_h, relation_h, user_h,
        tail_out, rel_out, item_out, user_out,
        idx_v, rows_v, sem):
    wid = lax.axis_index("s") * NC + lax.axis_index("c")

    def gather_chunks(idx_h, table_h, out_h, n_chunks, rows_per_w):
      idx_row_base = wid * (rows_per_w // IPC)
      out_base = wid * rows_per_w

      def body(c, _):
        pltpu.sync_copy(idx_h.at[pl.ds(idx_row_base + c * JPC, JPC)], idx_v)
        cps = [
            pltpu.async_copy(
                table_h.at[idx_v.at[j]],
                rows_v.at[pl.ds(j * IPC, IPC)], sem)
            for j in range(JPC)
        ]
        for cp in cps:
          cp.wait()
        pltpu.sync_copy(rows_v,
                        out_h.at[pl.ds(out_base + c * CHUNK, CHUNK)])
        return _

      lax.fori_loop(0, n_chunks, body, None)

    gather_chunks(tail_idx_h, entity_h, tail_out, NBR_CHUNKS, NBR_PER_W)
    gather_chunks(rel_idx_h, relation_h, rel_out, NBR_CHUNKS, NBR_PER_W)

    def gather_small(idx_h, table_h, out_h):
      jn = PAIR_PER_W // IPC
      pltpu.sync_copy(idx_h.at[pl.ds(wid * jn, jn)], idx_v.at[pl.ds(0, jn)])
      cps = [
          pltpu.async_copy(
              table_h.at[idx_v.at[j]],
              rows_v.at[pl.ds(j * IPC, IPC)], sem)
          for j in range(jn)
      ]
      for cp in cps:
        cp.wait()
      pltpu.sync_copy(rows_v.at[pl.ds(0, PAIR_PER_W)],
                      out_h.at[pl.ds(wid * PAIR_PER_W, PAIR_PER_W)])

    gather_small(item_idx_h, entity_h, item_out)
    gather_small(user_idx_h, user_h, user_out)

  return k(tail_idx, rel_idx, item_idx, user_idx,
           entity_mat, relation_mat, user_mat)


BLK = 512
GRID = B // BLK


def _dense_kernel(u_ref, it_ref, relf_ref, tailf_ref, wr_ref, wt_ref,
                  b_ref, y_ref, out_ref):
  u = u_ref[...]
  it = it_ref[...]
  x = (jnp.dot(relf_ref[...], wr_ref[...],
               preferred_element_type=jnp.float32)
       + jnp.dot(tailf_ref[...], wt_ref[...],
                 preferred_element_type=jnp.float32)
       + b_ref[...])
  fi = jax.nn.sigmoid(x)
  y = y_ref[0, 0, :]

  def bce_sum(logits):
    p = jax.nn.sigmoid(logits)
    p = jnp.clip(p, 1e-7, 1.0 - 1e-7)
    return jnp.sum(-(y * jnp.log(p) + (1.0 - y) * jnp.log1p(-p)))

  d1 = jnp.sum(u * it, axis=1)
  d2 = jnp.sum(u * fi, axis=1)
  partial = (bce_sum(d1) + bce_sum(d2)) * (1.0 / B)

  @pl.when(pl.program_id(0) == 0)
  def _():
    out_ref[...] = jnp.zeros_like(out_ref)

  out_ref[...] += partial.reshape(1, 1)


def _dense(user_rows, item_rows, rel_flat, tail_flat, wr_s, wt_s, b2, y3):
  return pl.pallas_call(
      _dense_kernel,
      grid=(GRID,),
      in_specs=[
          pl.BlockSpec((BLK, DIM), lambda i: (i, 0)),
          pl.BlockSpec((BLK, DIM), lambda i: (i, 0)),
          pl.BlockSpec((BLK, NN * DIM), lambda i: (i, 0)),
          pl.BlockSpec((BLK, NN * DIM), lambda i: (i, 0)),
          pl.BlockSpec((NN * DIM, DIM), lambda i: (0, 0)),
          pl.BlockSpec((NN * DIM, DIM), lambda i: (0, 0)),
          pl.BlockSpec((1, DIM), lambda i: (0, 0)),
          pl.BlockSpec((1, 1, BLK), lambda i: (i, 0, 0)),
      ],
      out_specs=pl.BlockSpec((1, 1), lambda i: (0, 0)),
      out_shape=jax.ShapeDtypeStruct((1, 1), jnp.float32),
  )(user_rows, item_rows, rel_flat, tail_flat, wr_s, wt_s, b2, y3)


def kernel(pairs, label, neighbor_relations, neighbor_tails,
           user_mat, entity_mat, relation_mat, W, b):
  user_idx = pairs[:, 0].reshape(B // IPC, IPC).astype(jnp.int32)
  item_idx = pairs[:, 1].reshape(B // IPC, IPC).astype(jnp.int32)
  tail_idx = neighbor_tails.reshape(TOT_NBR // IPC, IPC).astype(jnp.int32)
  rel_idx = neighbor_relations.reshape(TOT_NBR // IPC, IPC).astype(jnp.int32)

  tail_rows, rel_rows, item_rows, user_rows = _sc_gather(
      tail_idx, rel_idx, item_idx, user_idx,
      entity_mat, relation_mat, user_mat)

  w4 = W.reshape(DIM, NN, 2, DIM)
  wr_s = w4[:, :, 0, :].transpose(1, 2, 0).reshape(NN * DIM, DIM)
  wt_s = w4[:, :, 1, :].transpose(1, 2, 0).reshape(NN * DIM, DIM)

  rel_flat = rel_rows.reshape(B, NN * DIM)
  tail_flat = tail_rows.reshape(B, NN * DIM)
  b2 = b.reshape(1, DIM)
  y3 = label.reshape(GRID, 1, BLK)

  out = _dense(user_rows, item_rows, rel_flat, tail_flat, wr_s, wt_s, b2, y3)
  return out[0, 0]


# speedup vs baseline: 14.0659x; 1.1278x over previous
import functools

import jax
import jax.numpy as jnp
from jax import lax
from jax.experimental import pallas as pl
from jax.experimental.pallas import tpu as pltpu
from jax.experimental.pallas import tpu_sc as plsc

DIM = 32
NN = 20
B = 16384
NC = 2
NS = 16
NW = NC * NS
L = 16

IPC = 128
CHUNK = 1280
JPC = CHUNK // IPC

TOT_NBR = B * NN
NBR_PER_W = TOT_NBR // NW
NBR_CHUNKS = NBR_PER_W // CHUNK
PAIR_PER_W = B // NW

RT = B // 8
CT = NN * DIM // 128


def _sc_gather(tails1, rels1, users1, items1,
               entity_mat, relation_mat, user_mat):
  mesh = plsc.VectorSubcoreMesh(core_axis_name="c", subcore_axis_name="s")

  @functools.partial(
      pl.kernel, mesh=mesh,
      out_type=[
          jax.ShapeDtypeStruct((TOT_NBR, DIM), jnp.float32),
          jax.ShapeDtypeStruct((TOT_NBR, DIM), jnp.float32),
          jax.ShapeDtypeStruct((B, DIM), jnp.float32),
          jax.ShapeDtypeStruct((B, DIM), jnp.float32),
      ],
      scratch_types=[
          pltpu.VMEM((NBR_PER_W,), jnp.int32),
          pltpu.VMEM((CHUNK,), jnp.int32),
          pltpu.VMEM((CHUNK, DIM), jnp.float32),
          pltpu.SemaphoreType.DMA,
      ],
      compiler_params=pltpu.CompilerParams(
          use_tc_tiling_on_sc=False, needs_layout_passes=False),
  )
  def k(tails_h, rels_h, users_h, items_h, entity_h, relation_h, user_h,
        tail_out, rel_out, item_out, user_out,
        idx1d, flat_idx, rows_v, sem):
    wid = lax.axis_index("s") * NC + lax.axis_index("c")
    iota = lax.iota(jnp.int32, L)

    def gather_neighbors(idx_h, table_h, out_h):
      pltpu.sync_copy(idx_h.at[pl.ds(wid * NBR_PER_W, NBR_PER_W)], idx1d)

      def chunk_body(c, _):
        def flatten(s, _):
          dl = c * CHUNK + s * L + iota
          kk = dl & 3
          r = (dl >> 2) & 7
          q = dl >> 5
          b8 = (q * 52429) >> 18
          cc = q - b8 * 5
          src = (b8 * 8 + r) * NN + cc * 4 + kk
          flat_idx[pl.ds(s * L, L)] = plsc.load_gather(idx1d, [src])
          return _

        lax.fori_loop(0, CHUNK // L, flatten, None)
        cps = [
            pltpu.async_copy(
                table_h.at[flat_idx.at[pl.ds(j * IPC, IPC)]],
                rows_v.at[pl.ds(j * IPC, IPC)], sem)
            for j in range(JPC)
        ]
        for cp in cps:
          cp.wait()
        pltpu.sync_copy(
            rows_v, out_h.at[pl.ds(wid * NBR_PER_W + c * CHUNK, CHUNK)])
        return _

      lax.fori_loop(0, NBR_CHUNKS, chunk_body, None)

    gather_neighbors(tails_h, entity_h, tail_out)
    gather_neighbors(rels_h, relation_h, rel_out)

    def gather_pairs(pidx_h, table_h, out_h):
      pltpu.sync_copy(pidx_h.at[pl.ds(wid * PAIR_PER_W, PAIR_PER_W)],
                      flat_idx.at[pl.ds(0, PAIR_PER_W)])
      jn = PAIR_PER_W // IPC
      cps = [
          pltpu.async_copy(
              table_h.at[flat_idx.at[pl.ds(j * IPC, IPC)]],
              rows_v.at[pl.ds(j * IPC, IPC)], sem)
          for j in range(jn)
      ]
      for cp in cps:
        cp.wait()
      pltpu.sync_copy(rows_v.at[pl.ds(0, PAIR_PER_W)],
                      out_h.at[pl.ds(wid * PAIR_PER_W, PAIR_PER_W)])

    gather_pairs(items_h, entity_h, item_out)
    gather_pairs(users_h, user_h, user_out)

  return k(tails1, rels1, users1, items1,
           entity_mat, relation_mat, user_mat)


BLK = 512
GRID = B // BLK
BRT = BLK // 8


def _dense_kernel(u_ref, it_ref, rel4_ref, tail4_ref, wr_ref, wt_ref,
                  b_ref, y_ref, out_ref):
  u = u_ref[...]
  it = it_ref[...]
  x = b_ref[...]
  for g in range(CT):
    xr = rel4_ref[:, g].reshape(BLK, 128)
    xt = tail4_ref[:, g].reshape(BLK, 128)
    x = x + jnp.dot(xr, wr_ref[pl.ds(g * 128, 128), :],
                    preferred_element_type=jnp.float32)
    x = x + jnp.dot(xt, wt_ref[pl.ds(g * 128, 128), :],
                    preferred_element_type=jnp.float32)
  fi = jax.nn.sigmoid(x)
  y = y_ref[0, 0, :]

  def bce_sum(logits):
    p = jax.nn.sigmoid(logits)
    p = jnp.clip(p, 1e-7, 1.0 - 1e-7)
    return jnp.sum(-(y * jnp.log(p) + (1.0 - y) * jnp.log1p(-p)))

  d1 = jnp.sum(u * it, axis=1)
  d2 = jnp.sum(u * fi, axis=1)
  partial = (bce_sum(d1) + bce_sum(d2)) * (1.0 / B)

  @pl.when(pl.program_id(0) == 0)
  def _():
    out_ref[...] = jnp.zeros_like(out_ref)

  out_ref[...] += partial.reshape(1, 1)


def _dense(user_rows, item_rows, rel4, tail4, wr_s, wt_s, b2, y3):
  return pl.pallas_call(
      _dense_kernel,
      grid=(GRID,),
      in_specs=[
          pl.BlockSpec((BLK, DIM), lambda i: (i, 0)),
          pl.BlockSpec((BLK, DIM), lambda i: (i, 0)),
          pl.BlockSpec((BRT, CT, 8, 128), lambda i: (i, 0, 0, 0)),
          pl.BlockSpec((BRT, CT, 8, 128), lambda i: (i, 0, 0, 0)),
          pl.BlockSpec((NN * DIM, DIM), lambda i: (0, 0)),
          pl.BlockSpec((NN * DIM, DIM), lambda i: (0, 0)),
          pl.BlockSpec((1, DIM), lambda i: (0, 0)),
          pl.BlockSpec((1, 1, BLK), lambda i: (i, 0, 0)),
      ],
      out_specs=pl.BlockSpec((1, 1), lambda i: (0, 0)),
      out_shape=jax.ShapeDtypeStruct((1, 1), jnp.float32),
  )(user_rows, item_rows, rel4, tail4, wr_s, wt_s, b2, y3)


def kernel(pairs, label, neighbor_relations, neighbor_tails,
           user_mat, entity_mat, relation_mat, W, b):
  tail_rows, rel_rows, item_rows, user_rows = _sc_gather(
      neighbor_tails.reshape(-1).astype(jnp.int32),
      neighbor_relations.reshape(-1).astype(jnp.int32),
      pairs[:, 0].astype(jnp.int32), pairs[:, 1].astype(jnp.int32),
      entity_mat, relation_mat, user_mat)

  w4 = W.reshape(DIM, NN, 2, DIM)
  wr_s = w4[:, :, 0, :].transpose(1, 2, 0).reshape(NN * DIM, DIM)
  wt_s = w4[:, :, 1, :].transpose(1, 2, 0).reshape(NN * DIM, DIM)

  rel4 = rel_rows.reshape(RT, CT, 8, 128)
  tail4 = tail_rows.reshape(RT, CT, 8, 128)
  b2 = b.reshape(1, DIM)
  y3 = label.reshape(GRID, 1, BLK)

  out = _dense(user_rows, item_rows, rel4, tail4, wr_s, wt_s, b2, y3)
  return out[0, 0]
